# Initial kernel scaffold; baseline (speedup 1.0000x reference)
import jax
import jax.numpy as jnp
from jax.experimental import pallas as pl


def kernel(dec_input_emb, memory, cluster, params):
    raise NotImplementedError("write your pallas kernel here")


# speedup vs baseline: 2.3061x; 2.3061x over previous
import jax
import jax.numpy as jnp
from jax.experimental import pallas as pl

E = 8
K = 2
D = 1024
H = 16
DH = 64
FF = 4096
B = 128
LD = 128
LM = 256
TB_SA = 8
TB_CA = 4
BF = jnp.bfloat16
F32 = jnp.float32


def _gates_kernel(ml_ref, cl_ref, gates_ref, aux_ref):
    ml = ml_ref[...]
    cl = cl_ref[...]
    s = jax.lax.dot_general(ml, cl, (((1,), (1,)), ((), ())),
                            preferred_element_type=F32)
    s = s - jnp.max(s, axis=1, keepdims=True)
    p = jnp.exp(s)
    g = p / jnp.sum(p, axis=1, keepdims=True)
    g = jnp.where(g < 0.02, 0.0, g)
    idx = jax.lax.broadcasted_iota(jnp.int32, (B, E), 1)
    v1 = jnp.max(g, axis=1, keepdims=True)
    i1 = jnp.min(jnp.where(g == v1, idx, E), axis=1, keepdims=True)
    gm1 = jnp.where(idx == i1, -1.0, g)
    v2 = jnp.max(gm1, axis=1, keepdims=True)
    i2 = jnp.min(jnp.where(gm1 == v2, idx, E), axis=1, keepdims=True)
    keep = (idx == i1) | (idx == i2)
    gm = jnp.where(keep, g, 0.0)
    gates = gm / (jnp.sum(gm, axis=1, keepdims=True) + 1e-8)
    gates_ref[...] = gates
    gs = jnp.sum(gates, axis=0, keepdims=True)
    mu = jnp.mean(gs)
    var = jnp.sum((gs - mu) ** 2) / (E - 1)
    aux_ref[...] = jnp.reshape(var / (mu * mu + 1e-10) * 1e-12, (1, 1))


def _layer_norm(x, g, b):
    mu = jnp.mean(x, axis=-1, keepdims=True)
    var = jnp.mean((x - mu) ** 2, axis=-1, keepdims=True)
    return (x - mu) / jnp.sqrt(var + 1e-5) * g + b


def _attend(q, kv_k, kv_v, tb, lk):
    prod = q[:, None, :] * kv_k.reshape(tb, lk, D)
    sc = prod.reshape(tb, lk, H, DH).sum(-1) * 0.125
    sc = sc - jnp.max(sc, axis=1, keepdims=True)
    pw = jnp.exp(sc)
    pw = pw / jnp.sum(pw, axis=1, keepdims=True)
    pe = jnp.broadcast_to(pw[..., None], (tb, lk, H, DH)).reshape(tb, lk, D)
    return jnp.sum(pe * kv_v.reshape(tb, lk, D), axis=1)


def _sa_kernel(dec_ref, dl_ref, wq_ref, wk_ref, wv_ref, wo_ref, vec_ref,
               x1_ref):
    vec = vec_ref[0]
    bq, bk, bv, bo, g1, b1 = (vec[0], vec[1], vec[2], vec[3], vec[4], vec[5])
    d2 = dec_ref[...].reshape(TB_SA * LD, D)
    k = jnp.dot(d2, wk_ref[0], preferred_element_type=F32) + bk
    v = jnp.dot(d2, wv_ref[0], preferred_element_type=F32) + bv
    dl = dl_ref[...]
    q = jnp.dot(dl.astype(BF), wq_ref[0], preferred_element_type=F32) + bq
    o = _attend(q, k, v, TB_SA, LD)
    sa = jnp.dot(o.astype(BF), wo_ref[0], preferred_element_type=F32) + bo
    x1_ref[0] = _layer_norm(dl + sa, g1, b1)


def _ca_kernel(mem_ref, x1_ref, wq_ref, wk_ref, wv_ref, wo_ref, vec_ref,
               x2_ref):
    vec = vec_ref[0]
    bq, bk, bv, bo, g2, b2 = (vec[0], vec[1], vec[2], vec[3], vec[4], vec[5])
    m2 = mem_ref[...].reshape(TB_CA * LM, D)
    k = jnp.dot(m2, wk_ref[0], preferred_element_type=F32) + bk
    v = jnp.dot(m2, wv_ref[0], preferred_element_type=F32) + bv
    x1 = x1_ref[0]
    q = jnp.dot(x1.astype(BF), wq_ref[0], preferred_element_type=F32) + bq
    o = _attend(q, k, v, TB_CA, LM)
    ca = jnp.dot(o.astype(BF), wo_ref[0], preferred_element_type=F32) + bo
    x2_ref[0] = _layer_norm(x1 + ca, g2, b2)


def _ff_kernel(x2_ref, w1_ref, w2_ref, b1_ref, vec_ref, gt_ref, y_ref):
    e = pl.program_id(0)
    vec = vec_ref[0]
    b2, g3, b3 = vec[0], vec[1], vec[2]
    x2 = x2_ref[0]
    h = jnp.dot(x2.astype(BF), w1_ref[0], preferred_element_type=F32)
    h = jnp.maximum(h + b1_ref[0, 0], 0.0)
    f = jnp.dot(h.astype(BF), w2_ref[0], preferred_element_type=F32) + b2
    x3 = _layer_norm(x2 + f, g3, b3)
    contrib = x3 * gt_ref[0, 0][:, None]

    @pl.when(e == 0)
    def _():
        y_ref[...] = contrib

    @pl.when(e > 0)
    def _():
        y_ref[...] = y_ref[...] + contrib


def _pack_vec(p, names):
    rows = jnp.stack([p[n] for n in names], axis=1)
    pad = jnp.zeros((E, 8 - len(names), D), F32)
    return jnp.concatenate([rows, pad], axis=1)


def kernel(dec_input_emb, memory, cluster, params):
    p = params
    dec_bf = dec_input_emb.astype(BF)
    mem_bf = memory.astype(BF)
    dl = dec_input_emb[:, -1, :]
    ml = memory[:, -1, :]

    gates, aux = pl.pallas_call(
        _gates_kernel,
        out_shape=[jax.ShapeDtypeStruct((B, E), F32),
                   jax.ShapeDtypeStruct((1, 1), F32)],
    )(ml, cluster)

    wspec = pl.BlockSpec((1, D, D), lambda e, j: (e, 0, 0))
    vspec = pl.BlockSpec((1, 8, D), lambda e, j: (e, 0, 0))

    x1 = pl.pallas_call(
        _sa_kernel,
        grid=(E, B // TB_SA),
        in_specs=[
            pl.BlockSpec((TB_SA, LD, D), lambda e, j: (j, 0, 0)),
            pl.BlockSpec((TB_SA, D), lambda e, j: (j, 0)),
            wspec, wspec, wspec, wspec, vspec,
        ],
        out_specs=pl.BlockSpec((1, TB_SA, D), lambda e, j: (e, j, 0)),
        out_shape=jax.ShapeDtypeStruct((E, B, D), F32),
    )(dec_bf, dl,
      p['sa_wq'].astype(BF), p['sa_wk'].astype(BF), p['sa_wv'].astype(BF),
      p['sa_wo'].astype(BF),
      _pack_vec(p, ['sa_bq', 'sa_bk', 'sa_bv', 'sa_bo', 'ln1_g', 'ln1_b']))

    ncb = B // TB_CA
    x2 = pl.pallas_call(
        _ca_kernel,
        grid=(E, ncb),
        in_specs=[
            pl.BlockSpec((TB_CA, LM, D), lambda e, j: (j, 0, 0)),
            pl.BlockSpec((1, TB_CA, D), lambda e, j: (e * ncb + j, 0, 0)),
            wspec, wspec, wspec, wspec, vspec,
        ],
        out_specs=pl.BlockSpec((1, TB_CA, D), lambda e, j: (e * ncb + j, 0, 0)),
        out_shape=jax.ShapeDtypeStruct((E * ncb, TB_CA, D), F32),
    )(mem_bf, x1.reshape(E * ncb, TB_CA, D),
      p['ca_wq'].astype(BF), p['ca_wk'].astype(BF), p['ca_wv'].astype(BF),
      p['ca_wo'].astype(BF),
      _pack_vec(p, ['ca_bq', 'ca_bk', 'ca_bv', 'ca_bo', 'ln2_g', 'ln2_b']))

    gt = gates.T.reshape(E, 1, B)
    y = pl.pallas_call(
        _ff_kernel,
        grid=(E,),
        in_specs=[
            pl.BlockSpec((1, B, D), lambda e: (e, 0, 0)),
            pl.BlockSpec((1, D, FF), lambda e: (e, 0, 0)),
            pl.BlockSpec((1, FF, D), lambda e: (e, 0, 0)),
            pl.BlockSpec((1, 1, FF), lambda e: (e, 0, 0)),
            pl.BlockSpec((1, 8, D), lambda e: (e, 0, 0)),
            pl.BlockSpec((1, 1, B), lambda e: (e, 0, 0)),
        ],
        out_specs=pl.BlockSpec((B, D), lambda e: (0, 0)),
        out_shape=jax.ShapeDtypeStruct((B, D), F32),
    )(x2.reshape(E, B, D), p['ff_w1'].astype(BF), p['ff_w2'].astype(BF),
      p['ff_b1'].reshape(E, 1, FF),
      _pack_vec(p, ['ff_b2', 'ln3_g', 'ln3_b']),
      gt)

    return (y, aux[0, 0])


# speedup vs baseline: 4.1681x; 1.8075x over previous
import jax
import jax.numpy as jnp
from jax.experimental import pallas as pl

E = 8
K = 2
D = 1024
H = 16
DH = 64
FF = 4096
B = 128
LD = 128
LM = 256
TB_SA = 8
TB_CA = 4
BF = jnp.bfloat16
F32 = jnp.float32


def _gates_kernel(ml_ref, cl_ref, gates_ref, aux_ref):
    ml = ml_ref[...]
    cl = cl_ref[...]
    s = jax.lax.dot_general(ml, cl, (((1,), (1,)), ((), ())),
                            preferred_element_type=F32)
    s = s - jnp.max(s, axis=1, keepdims=True)
    p = jnp.exp(s)
    g = p / jnp.sum(p, axis=1, keepdims=True)
    g = jnp.where(g < 0.02, 0.0, g)
    idx = jax.lax.broadcasted_iota(jnp.int32, (B, E), 1)
    v1 = jnp.max(g, axis=1, keepdims=True)
    i1 = jnp.min(jnp.where(g == v1, idx, E), axis=1, keepdims=True)
    gm1 = jnp.where(idx == i1, -1.0, g)
    v2 = jnp.max(gm1, axis=1, keepdims=True)
    i2 = jnp.min(jnp.where(gm1 == v2, idx, E), axis=1, keepdims=True)
    keep = (idx == i1) | (idx == i2)
    gm = jnp.where(keep, g, 0.0)
    gates = gm / (jnp.sum(gm, axis=1, keepdims=True) + 1e-8)
    gates_ref[...] = gates
    gs = jnp.sum(gates, axis=0, keepdims=True)
    mu = jnp.mean(gs)
    var = jnp.sum((gs - mu) ** 2) / (E - 1)
    aux_ref[...] = jnp.reshape(var / (mu * mu + 1e-10) * 1e-12, (1, 1))


def _layer_norm(x, g, b):
    mu = jnp.mean(x, axis=-1, keepdims=True)
    var = jnp.mean((x - mu) ** 2, axis=-1, keepdims=True)
    return (x - mu) / jnp.sqrt(var + 1e-5) * g + b


def _attend(q, kv_k, kv_v, tb, lk, sel, selt):
    rows = tb * lk
    qb = jnp.broadcast_to(q[:, None, :], (tb, lk, D)).reshape(rows, D)
    prod = (qb * kv_k).astype(BF)
    sc = jnp.dot(prod, sel, preferred_element_type=F32).reshape(tb, lk, 128)
    sc = sc * 0.125
    sc = sc - jnp.max(sc, axis=1, keepdims=True)
    pw = jnp.exp(sc)
    pw = pw / jnp.sum(pw, axis=1, keepdims=True)
    pe = jnp.dot(pw.reshape(rows, 128).astype(BF), selt,
                 preferred_element_type=F32)
    return (pe * kv_v).reshape(tb, lk, D).sum(axis=1)


def _sa_kernel(dec_ref, dl_ref, wq_ref, wk_ref, wv_ref, wo_ref, vec_ref,
               sel_ref, selt_ref, x1_ref):
    vec = vec_ref[0]
    bq, bk, bv, bo, g1, b1 = (vec[0], vec[1], vec[2], vec[3], vec[4], vec[5])
    d2 = dec_ref[...].reshape(TB_SA * LD, D)
    k = jnp.dot(d2, wk_ref[0], preferred_element_type=F32) + bk
    v = jnp.dot(d2, wv_ref[0], preferred_element_type=F32) + bv
    dl = dl_ref[...]
    q = jnp.dot(dl.astype(BF), wq_ref[0], preferred_element_type=F32) + bq
    o = _attend(q, k, v, TB_SA, LD, sel_ref[...], selt_ref[...])
    sa = jnp.dot(o.astype(BF), wo_ref[0], preferred_element_type=F32) + bo
    x1_ref[0] = _layer_norm(dl + sa, g1, b1)


def _ca_kernel(mem_ref, x1_ref, wq_ref, wk_ref, wv_ref, wo_ref, vec_ref,
               sel_ref, selt_ref, x2_ref):
    vec = vec_ref[0]
    bq, bk, bv, bo, g2, b2 = (vec[0], vec[1], vec[2], vec[3], vec[4], vec[5])
    m2 = mem_ref[...].reshape(TB_CA * LM, D)
    k = jnp.dot(m2, wk_ref[0], preferred_element_type=F32) + bk
    v = jnp.dot(m2, wv_ref[0], preferred_element_type=F32) + bv
    x1 = x1_ref[0]
    q = jnp.dot(x1.astype(BF), wq_ref[0], preferred_element_type=F32) + bq
    o = _attend(q, k, v, TB_CA, LM, sel_ref[...], selt_ref[...])
    ca = jnp.dot(o.astype(BF), wo_ref[0], preferred_element_type=F32) + bo
    x2_ref[0] = _layer_norm(x1 + ca, g2, b2)


def _ff_kernel(x2_ref, w1_ref, w2_ref, b1_ref, vec_ref, gt_ref, y_ref):
    e = pl.program_id(0)
    vec = vec_ref[0]
    b2, g3, b3 = vec[0], vec[1], vec[2]
    x2 = x2_ref[0]
    h = jnp.dot(x2.astype(BF), w1_ref[0], preferred_element_type=F32)
    h = jnp.maximum(h + b1_ref[0, 0], 0.0)
    f = jnp.dot(h.astype(BF), w2_ref[0], preferred_element_type=F32) + b2
    x3 = _layer_norm(x2 + f, g3, b3)
    contrib = x3 * gt_ref[0, 0][:, None]

    @pl.when(e == 0)
    def _():
        y_ref[...] = contrib

    @pl.when(e > 0)
    def _():
        y_ref[...] = y_ref[...] + contrib


def _pack_vec(p, names):
    rows = jnp.stack([p[n] for n in names], axis=1)
    pad = jnp.zeros((E, 8 - len(names), D), F32)
    return jnp.concatenate([rows, pad], axis=1)


def kernel(dec_input_emb, memory, cluster, params):
    p = params
    dec_bf = dec_input_emb.astype(BF)
    mem_bf = memory.astype(BF)
    dl = dec_input_emb[:, -1, :]
    ml = memory[:, -1, :]

    gates, aux = pl.pallas_call(
        _gates_kernel,
        out_shape=[jax.ShapeDtypeStruct((B, E), F32),
                   jax.ShapeDtypeStruct((1, 1), F32)],
    )(ml, cluster)

    wspec = pl.BlockSpec((1, D, D), lambda e, j: (e, 0, 0))
    vspec = pl.BlockSpec((1, 8, D), lambda e, j: (e, 0, 0))
    sel = (jnp.arange(D)[:, None] // DH ==
           jnp.arange(128)[None, :]).astype(BF)
    selt = sel.T
    sspec = pl.BlockSpec((D, 128), lambda e, j: (0, 0))
    stspec = pl.BlockSpec((128, D), lambda e, j: (0, 0))

    x1 = pl.pallas_call(
        _sa_kernel,
        grid=(E, B // TB_SA),
        in_specs=[
            pl.BlockSpec((TB_SA, LD, D), lambda e, j: (j, 0, 0)),
            pl.BlockSpec((TB_SA, D), lambda e, j: (j, 0)),
            wspec, wspec, wspec, wspec, vspec, sspec, stspec,
        ],
        out_specs=pl.BlockSpec((1, TB_SA, D), lambda e, j: (e, j, 0)),
        out_shape=jax.ShapeDtypeStruct((E, B, D), F32),
    )(dec_bf, dl,
      p['sa_wq'].astype(BF), p['sa_wk'].astype(BF), p['sa_wv'].astype(BF),
      p['sa_wo'].astype(BF),
      _pack_vec(p, ['sa_bq', 'sa_bk', 'sa_bv', 'sa_bo', 'ln1_g', 'ln1_b']),
      sel, selt)

    ncb = B // TB_CA
    x2 = pl.pallas_call(
        _ca_kernel,
        grid=(E, ncb),
        in_specs=[
            pl.BlockSpec((TB_CA, LM, D), lambda e, j: (j, 0, 0)),
            pl.BlockSpec((1, TB_CA, D), lambda e, j: (e * ncb + j, 0, 0)),
            wspec, wspec, wspec, wspec, vspec, sspec, stspec,
        ],
        out_specs=pl.BlockSpec((1, TB_CA, D), lambda e, j: (e * ncb + j, 0, 0)),
        out_shape=jax.ShapeDtypeStruct((E * ncb, TB_CA, D), F32),
    )(mem_bf, x1.reshape(E * ncb, TB_CA, D),
      p['ca_wq'].astype(BF), p['ca_wk'].astype(BF), p['ca_wv'].astype(BF),
      p['ca_wo'].astype(BF),
      _pack_vec(p, ['ca_bq', 'ca_bk', 'ca_bv', 'ca_bo', 'ln2_g', 'ln2_b']),
      sel, selt)

    gt = gates.T.reshape(E, 1, B)
    y = pl.pallas_call(
        _ff_kernel,
        grid=(E,),
        in_specs=[
            pl.BlockSpec((1, B, D), lambda e: (e, 0, 0)),
            pl.BlockSpec((1, D, FF), lambda e: (e, 0, 0)),
            pl.BlockSpec((1, FF, D), lambda e: (e, 0, 0)),
            pl.BlockSpec((1, 1, FF), lambda e: (e, 0, 0)),
            pl.BlockSpec((1, 8, D), lambda e: (e, 0, 0)),
            pl.BlockSpec((1, 1, B), lambda e: (e, 0, 0)),
        ],
        out_specs=pl.BlockSpec((B, D), lambda e: (0, 0)),
        out_shape=jax.ShapeDtypeStruct((B, D), F32),
    )(x2.reshape(E, B, D), p['ff_w1'].astype(BF), p['ff_w2'].astype(BF),
      p['ff_b1'].reshape(E, 1, FF),
      _pack_vec(p, ['ff_b2', 'ln3_g', 'ln3_b']),
      gt)

    return (y, aux[0, 0])


# speedup vs baseline: 7.1281x; 1.7102x over previous
import jax
import jax.numpy as jnp
from jax.experimental import pallas as pl
from jax.experimental.pallas import tpu as pltpu

E = 8
D = 1024
DH = 64
FF = 4096
B = 128
LD = 128
LM = 256
BF = jnp.bfloat16
F32 = jnp.float32
I32 = jnp.int32


def _gates_kernel(ml_ref, cl_ref, aux_ref, ord_ref, cnt_ref, gslot_ref):
    ml = ml_ref[...]
    cl = cl_ref[...]
    s = jax.lax.dot_general(ml, cl, (((1,), (1,)), ((), ())),
                            preferred_element_type=F32)
    s = s - jnp.max(s, axis=1, keepdims=True)
    p = jnp.exp(s)
    g = p / jnp.sum(p, axis=1, keepdims=True)
    g = jnp.where(g < 0.02, 0.0, g)
    idx = jax.lax.broadcasted_iota(I32, (B, E), 1)
    v1 = jnp.max(g, axis=1, keepdims=True)
    i1 = jnp.min(jnp.where(g == v1, idx, E), axis=1, keepdims=True)
    gm1 = jnp.where(idx == i1, -1.0, g)
    v2 = jnp.max(gm1, axis=1, keepdims=True)
    i2 = jnp.min(jnp.where(gm1 == v2, idx, E), axis=1, keepdims=True)
    keep = (idx == i1) | (idx == i2)
    gm = jnp.where(keep, g, 0.0)
    gates = gm / (jnp.sum(gm, axis=1, keepdims=True) + 1e-8)
    gs = jnp.sum(gates, axis=0, keepdims=True)
    mu = jnp.mean(gs)
    var = jnp.sum((gs - mu) ** 2) / (E - 1)
    aux_ref[...] = jnp.reshape(var / (mu * mu + 1e-10) * 1e-12, (1, 1))

    maskf = (gates > 0.0).astype(F32)
    r_i = jax.lax.broadcasted_iota(I32, (B, B), 0)
    c_i = jax.lax.broadcasted_iota(I32, (B, B), 1)
    tri = (c_i <= r_i).astype(F32)
    pos = jnp.dot(tri, maskf, preferred_element_type=F32)
    cnt_ref[...] = pos[B - 1:B, :].astype(I32)
    posT = pos.T
    maskT = maskf.T
    gT = gates.T
    jrow = jax.lax.broadcasted_iota(I32, (E, B, B), 1)
    scol = jax.lax.broadcasted_iota(I32, (E, B, B), 2)
    cond = ((posT[:, None, :] == (jrow + 1).astype(F32)) &
            (maskT[:, None, :] > 0.0))
    ord_ref[...] = jnp.sum(jnp.where(cond, scol, 0), axis=2)
    gslot_ref[...] = jnp.sum(jnp.where(cond, gT[:, None, :], 0.0), axis=2)


def _prep_kernel(ord_ref, dl_ref, wq_ref, vec_ref, dlg_ref, qd_ref):
    ordv = ord_ref[0]
    scol = jax.lax.broadcasted_iota(I32, (B, B), 1)
    perm = (ordv.reshape(B, 1) == scol).astype(F32)
    dl = dl_ref[...]
    dlg = jnp.dot(perm, dl, preferred_element_type=F32)
    bq = vec_ref[0][0]
    qd = jnp.dot(dlg.astype(BF), wq_ref[0], preferred_element_type=F32) + bq
    dlg_ref[0] = dlg
    qd_ref[0] = qd


def _attend(q, kv_k, kv_v, tb, lk, sel, selt):
    rows = tb * lk
    qb = jnp.broadcast_to(q[:, None, :], (tb, lk, D)).reshape(rows, D)
    prod = (qb * kv_k).astype(BF)
    sc = jnp.dot(prod, sel, preferred_element_type=F32).reshape(tb, lk, 128)
    sc = sc * 0.125
    sc = sc - jnp.max(sc, axis=1, keepdims=True)
    pw = jnp.exp(sc)
    pw = pw / jnp.sum(pw, axis=1, keepdims=True)
    pe = jnp.dot(pw.reshape(rows, 128).astype(BF), selt,
                 preferred_element_type=F32)
    return (pe * kv_v).reshape(tb, lk, D).sum(axis=1)


def _make_core_kernel(lk):
    def _core(ord_ref, cnt_ref, seq_ref, q_ref, wk_ref, wv_ref, vec_ref,
              sel_ref, selt_ref, out_ref):
        j = pl.program_id(1)
        active = j < cnt_ref[0, pl.program_id(0)]

        @pl.when(active)
        def _():
            vec = vec_ref[0]
            bk, bv = vec[0], vec[1]
            s2 = seq_ref[0]
            k = jnp.dot(s2, wk_ref[0], preferred_element_type=F32) + bk
            v = jnp.dot(s2, wv_ref[0], preferred_element_type=F32) + bv
            out_ref[0] = _attend(q_ref[0], k, v, 1, lk,
                                 sel_ref[...], selt_ref[...])

        @pl.when(jnp.logical_not(active))
        def _():
            out_ref[0] = jnp.zeros((1, D), F32)

    return _core


def _oproj_kernel(attn_ref, dlg_ref, wo_ref, wq2_ref, vec_ref,
                  x1_ref, q2_ref):
    vec = vec_ref[0]
    bo, g1, b1, bq2 = vec[0], vec[1], vec[2], vec[3]
    o = jnp.dot(attn_ref[0].astype(BF), wo_ref[0],
                preferred_element_type=F32) + bo
    x1 = _layer_norm(dlg_ref[0] + o, g1, b1)
    x1_ref[0] = x1
    q2_ref[0] = jnp.dot(x1.astype(BF), wq2_ref[0],
                        preferred_element_type=F32) + bq2


def _layer_norm(x, g, b):
    mu = jnp.mean(x, axis=-1, keepdims=True)
    var = jnp.mean((x - mu) ** 2, axis=-1, keepdims=True)
    return (x - mu) / jnp.sqrt(var + 1e-5) * g + b


def _final_kernel(attn_ref, x1_ref, wo_ref, w1_ref, w2_ref, b1_ref, vec_ref,
                  gs_ref, ord_ref, y_ref):
    e = pl.program_id(0)
    vec = vec_ref[0]
    bo, g2, b2, fb2, g3, b3 = (vec[0], vec[1], vec[2], vec[3], vec[4],
                               vec[5])
    o2 = jnp.dot(attn_ref[0].astype(BF), wo_ref[0],
                 preferred_element_type=F32) + bo
    x2 = _layer_norm(x1_ref[0] + o2, g2, b2)
    h = jnp.dot(x2.astype(BF), w1_ref[0], preferred_element_type=F32)
    h = jnp.maximum(h + b1_ref[0, 0], 0.0)
    f = jnp.dot(h.astype(BF), w2_ref[0], preferred_element_type=F32) + fb2
    x3 = _layer_norm(x2 + f, g3, b3)
    contrib = x3 * gs_ref[0, 0][:, None]
    ordv = ord_ref[0]
    scol = jax.lax.broadcasted_iota(I32, (B, B), 1)
    perm = (ordv.reshape(B, 1) == scol).astype(F32)
    y_e = jax.lax.dot_general(perm, contrib, (((0,), (0,)), ((), ())),
                              preferred_element_type=F32)

    @pl.when(e == 0)
    def _():
        y_ref[...] = y_e

    @pl.when(e > 0)
    def _():
        y_ref[...] = y_ref[...] + y_e


def _pack_vec(p, names):
    rows = jnp.stack([p[n] for n in names], axis=1)
    pad = jnp.zeros((E, 8 - len(names), D), F32)
    return jnp.concatenate([rows, pad], axis=1)


def kernel(dec_input_emb, memory, cluster, params):
    p = params
    dec_bf = dec_input_emb.astype(BF)
    mem_bf = memory.astype(BF)
    dl = dec_input_emb[:, -1, :]
    ml = memory[:, -1, :]

    aux, order, count, gslot = pl.pallas_call(
        _gates_kernel,
        out_shape=[jax.ShapeDtypeStruct((1, 1), F32),
                   jax.ShapeDtypeStruct((E, B), I32),
                   jax.ShapeDtypeStruct((1, E), I32),
                   jax.ShapeDtypeStruct((E, B), F32)],
    )(ml, cluster)

    sel = (jnp.arange(D)[:, None] // DH ==
           jnp.arange(128)[None, :]).astype(BF)
    selt = sel.T
    ord3 = order.reshape(E, 1, B)

    wspec1 = pl.BlockSpec((1, D, D), lambda e: (e, 0, 0))
    vspec1 = pl.BlockSpec((1, 8, D), lambda e: (e, 0, 0))
    ospec1 = pl.BlockSpec((1, 1, B), lambda e: (e, 0, 0))
    aspec1 = pl.BlockSpec((1, B, D), lambda e: (e, 0, 0))

    dlg, qd = pl.pallas_call(
        _prep_kernel,
        grid=(E,),
        in_specs=[ospec1, pl.BlockSpec((B, D), lambda e: (0, 0)),
                  wspec1, vspec1],
        out_specs=[aspec1, aspec1],
        out_shape=[jax.ShapeDtypeStruct((E, B, D), F32),
                   jax.ShapeDtypeStruct((E, B, D), F32)],
    )(ord3, dl, p['sa_wq'].astype(BF), _pack_vec(p, ['sa_bq']))

    def _core_call(seq_bf, lseq, qsrc, wk, wv, vec):
        grid_spec = pltpu.PrefetchScalarGridSpec(
            num_scalar_prefetch=2,
            grid=(E, B),
            in_specs=[
                pl.BlockSpec((1, lseq, D),
                             lambda e, j, o_r, c_r: (o_r[e, j], 0, 0)),
                pl.BlockSpec((1, 1, D),
                             lambda e, j, o_r, c_r: (e * B + j, 0, 0)),
                pl.BlockSpec((1, D, D), lambda e, j, o_r, c_r: (e, 0, 0)),
                pl.BlockSpec((1, D, D), lambda e, j, o_r, c_r: (e, 0, 0)),
                pl.BlockSpec((1, 8, D), lambda e, j, o_r, c_r: (e, 0, 0)),
                pl.BlockSpec((D, 128), lambda e, j, o_r, c_r: (0, 0)),
                pl.BlockSpec((128, D), lambda e, j, o_r, c_r: (0, 0)),
            ],
            out_specs=pl.BlockSpec((1, 1, D),
                                   lambda e, j, o_r, c_r: (e * B + j, 0, 0)),
        )
        return pl.pallas_call(
            _make_core_kernel(lseq),
            grid_spec=grid_spec,
            out_shape=jax.ShapeDtypeStruct((E * B, 1, D), F32),
        )(order, count, seq_bf, qsrc.reshape(E * B, 1, D), wk, wv, vec,
          sel, selt)

    attn_sa = _core_call(dec_bf, LD, qd,
                         p['sa_wk'].astype(BF), p['sa_wv'].astype(BF),
                         _pack_vec(p, ['sa_bk', 'sa_bv']))

    x1, q2 = pl.pallas_call(
        _oproj_kernel,
        grid=(E,),
        in_specs=[aspec1, aspec1, wspec1, wspec1, vspec1],
        out_specs=[aspec1, aspec1],
        out_shape=[jax.ShapeDtypeStruct((E, B, D), F32),
                   jax.ShapeDtypeStruct((E, B, D), F32)],
    )(attn_sa.reshape(E, B, D), dlg, p['sa_wo'].astype(BF),
      p['ca_wq'].astype(BF),
      _pack_vec(p, ['sa_bo', 'ln1_g', 'ln1_b', 'ca_bq']))

    attn_ca = _core_call(mem_bf, LM, q2,
                         p['ca_wk'].astype(BF), p['ca_wv'].astype(BF),
                         _pack_vec(p, ['ca_bk', 'ca_bv']))

    y = pl.pallas_call(
        _final_kernel,
        grid=(E,),
        in_specs=[
            aspec1, aspec1, wspec1,
            pl.BlockSpec((1, D, FF), lambda e: (e, 0, 0)),
            pl.BlockSpec((1, FF, D), lambda e: (e, 0, 0)),
            pl.BlockSpec((1, 1, FF), lambda e: (e, 0, 0)),
            vspec1, ospec1, ospec1,
        ],
        out_specs=pl.BlockSpec((B, D), lambda e: (0, 0)),
        out_shape=jax.ShapeDtypeStruct((B, D), F32),
    )(attn_ca.reshape(E, B, D), x1, p['ca_wo'].astype(BF),
      p['ff_w1'].astype(BF), p['ff_w2'].astype(BF),
      p['ff_b1'].reshape(E, 1, FF),
      _pack_vec(p, ['ca_bo', 'ln2_g', 'ln2_b', 'ff_b2', 'ln3_g', 'ln3_b']),
      gslot.reshape(E, 1, B), ord3)

    return (y, aux[0, 0])
